# baseline (speedup 1.0000x reference)
import functools

import jax
import jax.numpy as jnp
from jax.experimental import pallas as pl
from jax.experimental.pallas import tpu as pltpu

N = 10000

BM = 200


def _bf(x):
    return x.astype(jnp.bfloat16)


def _stage1_kernel(a_ref, xk_ref, xi_ref, b1_ref, w1_ref, o_ref):
    agg = jnp.dot(_bf(a_ref[...]), _bf(xk_ref[...]),
                  preferred_element_type=jnp.float32)
    pre = agg - 3.0 * xi_ref[...]
    h = jnp.dot(_bf(pre), _bf(w1_ref[...]),
                preferred_element_type=jnp.float32) + b1_ref[...]
    o_ref[...] = jnp.maximum(h, 0.0)


def _stage2_kernel(a_ref, xk_ref, xi_ref, b2_ref, w2_ref, w3_ref, b3_ref,
                   o_ref):
    agg = jnp.dot(_bf(a_ref[...]), _bf(xk_ref[...]),
                  preferred_element_type=jnp.float32)
    pre = agg - 2.0 * xi_ref[...]
    h = jnp.dot(_bf(pre), _bf(w2_ref[...]),
                preferred_element_type=jnp.float32) + b2_ref[...]
    x2 = jnp.maximum(h, 0.0)
    o_ref[...] = jnp.dot(_bf(x2), _bf(w3_ref[...]),
                         preferred_element_type=jnp.float32) + b3_ref[...]


def kernel(A, X, W1, b1, W2, b2, W3, b3):
    n_i = N // BM
    D = X.shape[1]
    b1r = b1.reshape(1, -1)
    b2r = b2.reshape(1, -1)
    b3r = b3.reshape(1, 1)

    x1 = pl.pallas_call(
        _stage1_kernel,
        grid=(n_i,),
        in_specs=[
            pl.BlockSpec((BM, N), lambda i: (i, 0)),
            pl.BlockSpec((N, D), lambda i: (0, 0)),
            pl.BlockSpec((BM, D), lambda i: (i, 0)),
            pl.BlockSpec((1, W1.shape[1]), lambda i: (0, 0)),
            pl.BlockSpec((D, W1.shape[1]), lambda i: (0, 0)),
        ],
        out_specs=pl.BlockSpec((BM, W1.shape[1]), lambda i: (i, 0)),
        out_shape=jax.ShapeDtypeStruct((N, W1.shape[1]), jnp.float32),
        compiler_params=pltpu.CompilerParams(
            dimension_semantics=("parallel",),
        ),
    )(A, X, X, b1r, W1)

    out = pl.pallas_call(
        _stage2_kernel,
        grid=(n_i,),
        in_specs=[
            pl.BlockSpec((BM, N), lambda i: (i, 0)),
            pl.BlockSpec((N, 64), lambda i: (0, 0)),
            pl.BlockSpec((BM, 64), lambda i: (i, 0)),
            pl.BlockSpec((1, W2.shape[1]), lambda i: (0, 0)),
            pl.BlockSpec((64, W2.shape[1]), lambda i: (0, 0)),
            pl.BlockSpec((W2.shape[1], 1), lambda i: (0, 0)),
            pl.BlockSpec((1, 1), lambda i: (0, 0)),
        ],
        out_specs=pl.BlockSpec((BM, 1), lambda i: (i, 0)),
        out_shape=jax.ShapeDtypeStruct((N, 1), jnp.float32),
        compiler_params=pltpu.CompilerParams(
            dimension_semantics=("parallel",),
        ),
    )(A, x1, x1, b2r, W2, W3, b3r)
    return out


# speedup vs baseline: 1.0185x; 1.0185x over previous
import functools

import jax
import jax.numpy as jnp
from jax.experimental import pallas as pl
from jax.experimental.pallas import tpu as pltpu

N = 10000

BM = 400


def _bf(x):
    return x.astype(jnp.bfloat16)


def _stage1_kernel(a_ref, xk_ref, xi_ref, b1_ref, w1_ref, o_ref):
    agg = jnp.dot(_bf(a_ref[...]), _bf(xk_ref[...]),
                  preferred_element_type=jnp.float32)
    pre = agg - 3.0 * xi_ref[...]
    h = jnp.dot(_bf(pre), _bf(w1_ref[...]),
                preferred_element_type=jnp.float32) + b1_ref[...]
    o_ref[...] = jnp.maximum(h, 0.0)


def _stage2_kernel(a_ref, xk_ref, xi_ref, b2_ref, w2_ref, w3_ref, b3_ref,
                   o_ref):
    agg = jnp.dot(_bf(a_ref[...]), _bf(xk_ref[...]),
                  preferred_element_type=jnp.float32)
    pre = agg - 2.0 * xi_ref[...]
    h = jnp.dot(_bf(pre), _bf(w2_ref[...]),
                preferred_element_type=jnp.float32) + b2_ref[...]
    x2 = jnp.maximum(h, 0.0)
    o_ref[...] = jnp.dot(_bf(x2), _bf(w3_ref[...]),
                         preferred_element_type=jnp.float32) + b3_ref[...]


def kernel(A, X, W1, b1, W2, b2, W3, b3):
    n_i = N // BM
    D = X.shape[1]
    b1r = b1.reshape(1, -1)
    b2r = b2.reshape(1, -1)
    b3r = b3.reshape(1, 1)

    x1 = pl.pallas_call(
        _stage1_kernel,
        grid=(n_i,),
        in_specs=[
            pl.BlockSpec((BM, N), lambda i: (i, 0)),
            pl.BlockSpec((N, D), lambda i: (0, 0)),
            pl.BlockSpec((BM, D), lambda i: (i, 0)),
            pl.BlockSpec((1, W1.shape[1]), lambda i: (0, 0)),
            pl.BlockSpec((D, W1.shape[1]), lambda i: (0, 0)),
        ],
        out_specs=pl.BlockSpec((BM, W1.shape[1]), lambda i: (i, 0)),
        out_shape=jax.ShapeDtypeStruct((N, W1.shape[1]), jnp.float32),
        compiler_params=pltpu.CompilerParams(
            dimension_semantics=("parallel",),
        ),
    )(A, X, X, b1r, W1)

    out = pl.pallas_call(
        _stage2_kernel,
        grid=(n_i,),
        in_specs=[
            pl.BlockSpec((BM, N), lambda i: (i, 0)),
            pl.BlockSpec((N, 64), lambda i: (0, 0)),
            pl.BlockSpec((BM, 64), lambda i: (i, 0)),
            pl.BlockSpec((1, W2.shape[1]), lambda i: (0, 0)),
            pl.BlockSpec((64, W2.shape[1]), lambda i: (0, 0)),
            pl.BlockSpec((W2.shape[1], 1), lambda i: (0, 0)),
            pl.BlockSpec((1, 1), lambda i: (0, 0)),
        ],
        out_specs=pl.BlockSpec((BM, 1), lambda i: (i, 0)),
        out_shape=jax.ShapeDtypeStruct((N, 1), jnp.float32),
        compiler_params=pltpu.CompilerParams(
            dimension_semantics=("parallel",),
        ),
    )(A, x1, x1, b2r, W2, W3, b3r)
    return out


# speedup vs baseline: 1.0473x; 1.0283x over previous
import functools

import jax
import jax.numpy as jnp
from jax.experimental import pallas as pl
from jax.experimental.pallas import tpu as pltpu

N = 10000

BM = 400
N_I = N // BM


def _bf(x):
    return x.astype(jnp.bfloat16)


def _fused_kernel(a_ref, xbf_ref, xi_ref, b1_ref, w1_ref, b2_ref, w2_ref,
                  w3_ref, b3_ref, o_ref, x1f_ref, x1bf_ref):
    s = pl.program_id(0)

    @pl.when(s < N_I)
    def _stage1():
        agg = jnp.dot(_bf(a_ref[...]), xbf_ref[...],
                      preferred_element_type=jnp.float32)
        pre = agg - 3.0 * xi_ref[...]
        h = jnp.dot(_bf(pre), _bf(w1_ref[...]),
                    preferred_element_type=jnp.float32) + b1_ref[...]
        x1 = jnp.maximum(h, 0.0)
        x1f_ref[pl.ds(s * BM, BM), :] = x1
        x1bf_ref[pl.ds(s * BM, BM), :] = _bf(x1)

    @pl.when(s >= N_I)
    def _stage2():
        i = s - N_I
        agg = jnp.dot(_bf(a_ref[...]), x1bf_ref[...],
                      preferred_element_type=jnp.float32)
        pre = agg - 2.0 * x1f_ref[pl.ds(i * BM, BM), :]
        h = jnp.dot(_bf(pre), _bf(w2_ref[...]),
                    preferred_element_type=jnp.float32) + b2_ref[...]
        x2 = jnp.maximum(h, 0.0)
        o_ref[...] = jnp.dot(_bf(x2), _bf(w3_ref[...]),
                             preferred_element_type=jnp.float32) + b3_ref[...]


def kernel(A, X, W1, b1, W2, b2, W3, b3):
    D = X.shape[1]
    H1 = W1.shape[1]
    H2 = W2.shape[1]
    x_bf = X.astype(jnp.bfloat16)

    return pl.pallas_call(
        _fused_kernel,
        grid=(2 * N_I,),
        in_specs=[
            pl.BlockSpec((BM, N), lambda s: (s % N_I, 0)),
            pl.BlockSpec((N, D), lambda s: (0, 0)),
            pl.BlockSpec((BM, D), lambda s: (s % N_I, 0)),
            pl.BlockSpec((1, H1), lambda s: (0, 0)),
            pl.BlockSpec((D, H1), lambda s: (0, 0)),
            pl.BlockSpec((1, H2), lambda s: (0, 0)),
            pl.BlockSpec((H1, H2), lambda s: (0, 0)),
            pl.BlockSpec((H2, 1), lambda s: (0, 0)),
            pl.BlockSpec((1, 1), lambda s: (0, 0)),
        ],
        out_specs=pl.BlockSpec((BM, 1), lambda s: (s % N_I, 0)),
        out_shape=jax.ShapeDtypeStruct((N, 1), jnp.float32),
        scratch_shapes=[
            pltpu.VMEM((N, H1), jnp.float32),
            pltpu.VMEM((N, H1), jnp.bfloat16),
        ],
        compiler_params=pltpu.CompilerParams(
            dimension_semantics=("arbitrary",),
        ),
    )(A, x_bf, X, b1.reshape(1, -1), W1, b2.reshape(1, -1), W2, W3,
      b3.reshape(1, 1))
